# baseline (speedup 1.0000x reference)
import functools

import jax
import jax.numpy as jnp
from jax import lax
from jax.experimental import pallas as pl
from jax.experimental.pallas import tpu as pltpu
from jax.experimental.pallas import tpu_sc as plsc

N_NODES = 100000
D_FEAT = 128
NCOL = D_FEAT // 16
NUM_GRAPHS = 256
NUM_GLOBAL = 16
CHUNK = 400
SUB = 100
NSUB = CHUNK // SUB
NGRP = CHUNK // 16
NCHUNK = N_NODES // CHUNK
NW = 32
MAX_CHUNKS_PER_W = -(-NCHUNK // NW)
NPART = 16
DUMMY = NUM_GRAPHS
CW = 16


def _sc_segment_sums(x, batch_i32):
    mesh = plsc.VectorSubcoreMesh(core_axis_name="c", subcore_axis_name="s")

    @functools.partial(
        pl.kernel,
        mesh=mesh,
        out_type=(
            jax.ShapeDtypeStruct((2 * NUM_GRAPHS, D_FEAT), jnp.float32),
            jax.ShapeDtypeStruct((2 * NUM_GRAPHS, CW), jnp.float32),
        ),
        scratch_types=(
            pltpu.VMEM((CHUNK, D_FEAT), jnp.float32),
            pltpu.VMEM((CHUNK, D_FEAT), jnp.float32),
            pltpu.VMEM((NSUB, SUB), jnp.int32),
            pltpu.VMEM((NSUB, SUB), jnp.int32),
            pltpu.VMEM((NPART, D_FEAT), jnp.float32),
            pltpu.VMEM((NPART,), jnp.int32),
            pltpu.VMEM((NPART, CW), jnp.float32),
            pltpu.VMEM((SUB, CW), jnp.float32),
            pltpu.VMEM((16, D_FEAT), jnp.float32),
            pltpu.VMEM((16, CW), jnp.float32),
            pltpu.VMEM_SHARED((NUM_GRAPHS + 1, D_FEAT), jnp.float32),
            pltpu.VMEM_SHARED((NUM_GRAPHS + 1, CW), jnp.float32),
            pltpu.SemaphoreType.DMA,
            pltpu.SemaphoreType.DMA,
            pltpu.SemaphoreType.DMA,
            pltpu.SemaphoreType.DMA,
        ),
        compiler_params=pltpu.CompilerParams(use_tc_tiling_on_sc=False,
                                             needs_layout_passes=False),
    )
    def sc_kernel(x_hbm, b_hbm, sums_hbm, cnts_hbm,
                  xbuf0, xbuf1, idxbuf0, idxbuf1, parts, pids, pcnts, onesbuf,
                  bounce_d, bounce_c, acc_sh, cnt_sh,
                  semx0, semx1, semi0, semi1):
        c = lax.axis_index("c")
        s = lax.axis_index("s")
        w = s * 2 + c
        xbufs = (xbuf0, xbuf1)
        idxbufs = (idxbuf0, idxbuf1)
        semxs = (semx0, semx1)
        semis = (semi0, semi1)
        lanes = lax.iota(jnp.int32, 16)

        zeros16 = jnp.zeros((16,), jnp.float32)
        ones16 = jnp.ones((16,), jnp.float32)
        for r in range(16):
            for col in range(NCOL):
                bounce_d[r, pl.ds(col * 16, 16)] = zeros16
            bounce_c[r, :] = zeros16
        for r in range(SUB):
            onesbuf[r, :] = ones16

        pltpu.sync_copy(bounce_d, acc_sh.at[pl.ds(s * 16, 16)])
        pltpu.sync_copy(bounce_c, cnt_sh.at[pl.ds(s * 16, 16)])
        plsc.subcore_barrier()

        def fetch(j, slot):
            i = w + NW * j
            pltpu.async_copy(x_hbm.at[pl.ds(i * CHUNK, CHUNK)],
                             xbufs[slot], semxs[slot])
            pltpu.async_copy(b_hbm.at[pl.ds(i * NSUB, NSUB)],
                             idxbufs[slot], semis[slot])

        def wait_fetch(j, slot):
            i = w + NW * j
            pltpu.make_async_copy(x_hbm.at[pl.ds(i * CHUNK, CHUNK)],
                                  xbufs[slot], semxs[slot]).wait()
            pltpu.make_async_copy(b_hbm.at[pl.ds(i * NSUB, NSUB)],
                                  idxbufs[slot], semis[slot]).wait()

        def flush_run(accs8, run_id, run_cnt, slot_idx):
            prow = jnp.full((16,), slot_idx, jnp.int32)
            for col in range(NCOL):
                plsc.store_scatter(parts, [prow, lanes + col * 16],
                                   accs8[col])
            plsc.store_scatter(pids, [prow], jnp.full((16,), run_id,
                                                      jnp.int32))
            plsc.store_scatter(pcnts, [prow, lanes],
                               jnp.full((16,), run_cnt.astype(jnp.float32)))

        def process_chunk(slot):
            xb = xbufs[slot]
            ib = idxbufs[slot]
            pids[:] = jnp.full((16,), DUMMY, jnp.int32)

            def id_group(g):
                flat = g * 16 + lanes
                return plsc.load_gather(ib, [flat // SUB, flat % SUB])

            def load_row(i):
                rid = jnp.full((16,), i, jnp.int32)
                return [plsc.load_gather(xb, [rid, lanes + col * 16])
                        for col in range(NCOL)]

            id0 = id_group(0)[0]

            def grp_body(g, carry):
                accs = list(carry[:NCOL])
                cur_id, p, cnt = carry[NCOL], carry[NCOL + 1], carry[NCOL + 2]
                idvec = id_group(g)
                clean = jnp.logical_not(
                    jnp.any(idvec != jnp.full((16,), cur_id, jnp.int32)))

                def fast(*args):
                    accs = list(args[:NCOL])
                    cur_id, p, cnt = args[NCOL:]
                    for l in range(16):
                        row = load_row(g * 16 + l)
                        accs = [accs[col] + row[col] for col in range(NCOL)]
                    return tuple(accs) + (cur_id, p, cnt + 16)

                def slow(*args):
                    accs = list(args[:NCOL])
                    cur_id, p, cnt = args[NCOL:]
                    for l in range(16):
                        new_id = idvec[l]
                        row = load_row(g * 16 + l)
                        is_new = jnp.logical_and(new_id != cur_id, cnt > 0)

                        def do_flush(a=tuple(accs), ci=cur_id, pp=p, cc=cnt):
                            flush_run(a, ci, cc, jnp.minimum(pp, NPART - 1))
                            return jnp.int32(1)

                        bump = lax.cond(is_new, do_flush,
                                        lambda: jnp.int32(0))
                        accs = [jnp.where(is_new, row[col],
                                          accs[col] + row[col])
                                for col in range(NCOL)]
                        p = p + bump
                        cnt = jnp.where(is_new, 1, cnt + 1)
                        cur_id = new_id
                    return tuple(accs) + (cur_id, p, cnt)

                return lax.cond(clean, fast, slow, *accs, cur_id, p, cnt)

            init = tuple(jnp.zeros((16,), jnp.float32)
                         for _ in range(NCOL)) + (id0, jnp.int32(0),
                                                  jnp.int32(0))
            carry = lax.fori_loop(0, NGRP, grp_body, init, unroll=False)
            accs = carry[:NCOL]
            cur_id, p, cnt = carry[NCOL], carry[NCOL + 1], carry[NCOL + 2]
            flush_run(accs, cur_id, cnt, jnp.minimum(p, NPART - 1))

            def scatter_parts():
                pltpu.sync_copy(parts, acc_sh.at[pids], add=True)
                pltpu.sync_copy(pcnts, cnt_sh.at[pids], add=True)

            def scatter_raw():
                for k in range(NSUB):
                    pltpu.sync_copy(xb.at[pl.ds(k * SUB, SUB)],
                                    acc_sh.at[ib.at[k]], add=True)
                    pltpu.sync_copy(onesbuf, cnt_sh.at[ib.at[k]], add=True)

            lax.cond(p < NPART, scatter_parts, scatter_raw)

        fetch(0, 0)
        for j in range(MAX_CHUNKS_PER_W):
            slot = j % 2
            i = w + NW * j

            if j + 1 < MAX_CHUNKS_PER_W:
                @pl.when(w + NW * (j + 1) < NCHUNK)
                def _():
                    fetch(j + 1, 1 - slot)

            @pl.when(i < NCHUNK)
            def _():
                wait_fetch(j, slot)
                process_chunk(slot)

        plsc.subcore_barrier()

        out_row = c * NUM_GRAPHS + s * 16
        pltpu.sync_copy(acc_sh.at[pl.ds(s * 16, 16)], bounce_d)
        pltpu.sync_copy(bounce_d, sums_hbm.at[pl.ds(out_row, 16)])
        pltpu.sync_copy(cnt_sh.at[pl.ds(s * 16, 16)], bounce_c)
        pltpu.sync_copy(bounce_c, cnts_hbm.at[pl.ds(out_row, 16)])

    return sc_kernel(x, batch_i32)


def _tc_mlp(sums2, cnts2, u, W1, b1, W2, b2):
    g = NUM_GRAPHS

    def body(s_ref, c_ref, u_ref, w1_ref, b1_ref, w2_ref, b2_ref, o_ref):
        sums = s_ref[0:g, :] + s_ref[g:2 * g, :]
        counts = c_ref[0:g, 0:1] + c_ref[g:2 * g, 0:1]
        mean = sums / jnp.maximum(counts, 1.0)
        h = (jnp.dot(u_ref[:], w1_ref[0:NUM_GLOBAL, :],
                     preferred_element_type=jnp.float32)
             + jnp.dot(mean, w1_ref[NUM_GLOBAL:, :],
                       preferred_element_type=jnp.float32)
             + b1_ref[:])
        h = jnp.maximum(h, 0.0)
        o_ref[:] = jnp.dot(h, w2_ref[:],
                           preferred_element_type=jnp.float32) + b2_ref[:]

    return pl.pallas_call(
        body,
        out_shape=jax.ShapeDtypeStruct((g, W2.shape[1]), jnp.float32),
    )(sums2, cnts2, u, W1, b1.reshape(1, -1), W2, b2.reshape(1, -1))


def kernel(x, edge_index, edge_attr, u, batch, W1, b1, W2, b2):
    del edge_index, edge_attr
    batch_2d = batch.astype(jnp.int32).reshape(NCHUNK * NSUB, SUB)
    sums2, cnts2 = _sc_segment_sums(x, batch_2d)
    return _tc_mlp(sums2, cnts2, u, W1, b1, W2, b2)


# speedup vs baseline: 1.2370x; 1.2370x over previous
import functools

import jax
import jax.numpy as jnp
from jax import lax
from jax.experimental import pallas as pl
from jax.experimental.pallas import tpu as pltpu
from jax.experimental.pallas import tpu_sc as plsc

N_NODES = 100000
D_FEAT = 128
NUM_GRAPHS = 256
NUM_GLOBAL = 16
CHUNK = 400
SUB = 100
NSUB = CHUNK // SUB
NCHUNK = N_NODES // CHUNK
NW = 32
MAX_CHUNKS_PER_W = -(-NCHUNK // NW)
CW = 16


def _sc_segment_sums(x, batch_i32):
    mesh = plsc.VectorSubcoreMesh(core_axis_name="c", subcore_axis_name="s")

    @functools.partial(
        pl.kernel,
        mesh=mesh,
        out_type=(
            jax.ShapeDtypeStruct((2 * NUM_GRAPHS, D_FEAT), jnp.float32),
            jax.ShapeDtypeStruct((2 * NUM_GRAPHS, CW), jnp.float32),
        ),
        scratch_types=(
            pltpu.VMEM((CHUNK, D_FEAT), jnp.float32),
            pltpu.VMEM((CHUNK, D_FEAT), jnp.float32),
            pltpu.VMEM((NSUB, SUB), jnp.int32),
            pltpu.VMEM((NSUB, SUB), jnp.int32),
            pltpu.VMEM((SUB, CW), jnp.float32),
            pltpu.VMEM((16, D_FEAT), jnp.float32),
            pltpu.VMEM((16, CW), jnp.float32),
            pltpu.VMEM_SHARED((NUM_GRAPHS, D_FEAT), jnp.float32),
            pltpu.VMEM_SHARED((NUM_GRAPHS, CW), jnp.float32),
            pltpu.SemaphoreType.DMA,
            pltpu.SemaphoreType.DMA,
            pltpu.SemaphoreType.DMA,
            pltpu.SemaphoreType.DMA,
            pltpu.SemaphoreType.DMA,
            pltpu.SemaphoreType.DMA,
        ),
        compiler_params=pltpu.CompilerParams(use_tc_tiling_on_sc=False),
    )
    def sc_kernel(x_hbm, b_hbm, sums_hbm, cnts_hbm,
                  xbuf0, xbuf1, idxbuf0, idxbuf1, onesbuf, bounce_d, bounce_c,
                  acc_sh, cnt_sh, semx0, semx1, semi0, semi1, sems0, sems1):
        c = lax.axis_index("c")
        s = lax.axis_index("s")
        w = s * 2 + c
        xbufs = (xbuf0, xbuf1)
        idxbufs = (idxbuf0, idxbuf1)
        semxs = (semx0, semx1)
        semis = (semi0, semi1)
        semss = (sems0, sems1)

        zeros16 = jnp.zeros((16,), jnp.float32)
        ones16 = jnp.ones((16,), jnp.float32)
        for r in range(16):
            for col in range(D_FEAT // 16):
                bounce_d[r, pl.ds(col * 16, 16)] = zeros16
            bounce_c[r, :] = zeros16
        for r in range(SUB):
            onesbuf[r, :] = ones16

        pltpu.sync_copy(bounce_d, acc_sh.at[pl.ds(s * 16, 16)])
        pltpu.sync_copy(bounce_c, cnt_sh.at[pl.ds(s * 16, 16)])
        plsc.subcore_barrier()

        def fetch(j, slot):
            i = w + NW * j
            pltpu.async_copy(x_hbm.at[pl.ds(i * CHUNK, CHUNK)],
                             xbufs[slot], semxs[slot])
            pltpu.async_copy(b_hbm.at[pl.ds(i * NSUB, NSUB)],
                             idxbufs[slot], semis[slot])

        def wait_fetch(j, slot):
            i = w + NW * j
            pltpu.make_async_copy(x_hbm.at[pl.ds(i * CHUNK, CHUNK)],
                                  xbufs[slot], semxs[slot]).wait()
            pltpu.make_async_copy(b_hbm.at[pl.ds(i * NSUB, NSUB)],
                                  idxbufs[slot], semis[slot]).wait()

        def issue_scatters(slot):
            for k in range(NSUB):
                pltpu.async_copy(xbufs[slot].at[pl.ds(k * SUB, SUB)],
                                 acc_sh.at[idxbufs[slot].at[k]],
                                 semss[slot], add=True)
                pltpu.async_copy(onesbuf, cnt_sh.at[idxbufs[slot].at[k]],
                                 semss[slot], add=True)

        def wait_scatters(slot):
            for k in range(NSUB):
                pltpu.make_async_copy(xbufs[slot].at[pl.ds(k * SUB, SUB)],
                                      acc_sh.at[idxbufs[slot].at[k]],
                                      semss[slot]).wait()
                pltpu.make_async_copy(onesbuf, cnt_sh.at[idxbufs[slot].at[k]],
                                      semss[slot]).wait()

        fetch(0, 0)
        for j in range(MAX_CHUNKS_PER_W):
            slot = j % 2
            i = w + NW * j

            @pl.when(i < NCHUNK)
            def _():
                wait_fetch(j, slot)
                issue_scatters(slot)

            if j >= 1:
                @pl.when(w + NW * (j - 1) < NCHUNK)
                def _():
                    wait_scatters(1 - slot)

            if j + 1 < MAX_CHUNKS_PER_W:
                @pl.when(w + NW * (j + 1) < NCHUNK)
                def _():
                    fetch(j + 1, 1 - slot)

        last = MAX_CHUNKS_PER_W - 1

        @pl.when(w + NW * last < NCHUNK)
        def _():
            wait_scatters(last % 2)

        plsc.subcore_barrier()

        out_row = c * NUM_GRAPHS + s * 16
        pltpu.sync_copy(acc_sh.at[pl.ds(s * 16, 16)], bounce_d)
        pltpu.sync_copy(bounce_d, sums_hbm.at[pl.ds(out_row, 16)])
        pltpu.sync_copy(cnt_sh.at[pl.ds(s * 16, 16)], bounce_c)
        pltpu.sync_copy(bounce_c, cnts_hbm.at[pl.ds(out_row, 16)])

    return sc_kernel(x, batch_i32)


def _tc_mlp(sums2, cnts2, u, W1, b1, W2, b2):
    g = NUM_GRAPHS

    def body(s_ref, c_ref, u_ref, w1_ref, b1_ref, w2_ref, b2_ref, o_ref):
        sums = s_ref[0:g, :] + s_ref[g:2 * g, :]
        counts = c_ref[0:g, 0:1] + c_ref[g:2 * g, 0:1]
        mean = sums / jnp.maximum(counts, 1.0)
        h = (jnp.dot(u_ref[:], w1_ref[0:NUM_GLOBAL, :],
                     preferred_element_type=jnp.float32)
             + jnp.dot(mean, w1_ref[NUM_GLOBAL:, :],
                       preferred_element_type=jnp.float32)
             + b1_ref[:])
        h = jnp.maximum(h, 0.0)
        o_ref[:] = jnp.dot(h, w2_ref[:],
                           preferred_element_type=jnp.float32) + b2_ref[:]

    return pl.pallas_call(
        body,
        out_shape=jax.ShapeDtypeStruct((g, W2.shape[1]), jnp.float32),
    )(sums2, cnts2, u, W1, b1.reshape(1, -1), W2, b2.reshape(1, -1))


def kernel(x, edge_index, edge_attr, u, batch, W1, b1, W2, b2):
    del edge_index, edge_attr
    batch_2d = batch.astype(jnp.int32).reshape(NCHUNK * NSUB, SUB)
    sums2, cnts2 = _sc_segment_sums(x, batch_2d)
    return _tc_mlp(sums2, cnts2, u, W1, b1, W2, b2)
